# Initial kernel scaffold; baseline (speedup 1.0000x reference)
import jax
import jax.numpy as jnp
from jax.experimental import pallas as pl


def kernel(char_ids, img_table, w3d1, b3d1, w3d2, b3d2, w2d, b2d, wfc, bfc):
    raise NotImplementedError("write your pallas kernel here")


# speedup vs baseline: 1.3807x; 1.3807x over previous
import numpy as np
import jax
import jax.numpy as jnp
from jax.experimental import pallas as pl
from jax.experimental.pallas import tpu as pltpu

_F32 = jnp.float32


def _params():
    return pltpu.CompilerParams(
        dimension_semantics=("parallel",),
        vmem_limit_bytes=48 * 1024 * 1024,
    )


def _stage_a(char_ids, img_table, w3d1, b3d1, w3d2, b3d2):
    B, N = char_ids.shape
    V = img_table.shape[0]
    table = img_table.reshape(V, 50, 50)
    D, H, W = N, 50, 50
    Dp, Hp, Wp = D + 2, H + 2, W + 2

    ids = char_ids.astype(jnp.int32)
    valid = ((ids >= 0) & (ids < V)).astype(jnp.int32)
    safe = jnp.clip(ids, 0, V - 1)

    w1 = w3d1.reshape(-1).astype(_F32)
    b1 = b3d1.reshape(-1).astype(_F32)
    w2 = w3d2.reshape(-1).astype(_F32)
    b2 = b3d2.reshape(-1).astype(_F32)

    def body(ids_ref, valid_ref, *refs):
        glyph_refs = refs[:N]
        w1_ref, b1_ref, w2_ref, b2_ref, o_ref, xp, h1p = refs[N:]
        b = pl.program_id(0)

        xp[...] = jnp.zeros((Dp, Hp, Wp), _F32)
        for n in range(N):
            v = valid_ref[b, n].astype(_F32)
            xp[n + 1, 1:1 + H, 1:1 + W] = glyph_refs[n][0] * v

        h1p[...] = jnp.zeros((4, Dp, Hp, Wp), _F32)
        for co in range(4):
            acc = None
            for kd in range(3):
                for kh in range(3):
                    for kw in range(3):
                        idx = ((co * 3 + kd) * 3 + kh) * 3 + kw
                        t = xp[kd:kd + D, kh:kh + H, kw:kw + W] * w1_ref[idx]
                        acc = t if acc is None else acc + t
            h1p[co, 1:1 + D, 1:1 + H, 1:1 + W] = acc + b1_ref[co]

        acc = None
        for ci in range(4):
            for kd in range(3):
                for kh in range(3):
                    for kw in range(3):
                        idx = ((ci * 3 + kd) * 3 + kh) * 3 + kw
                        t = h1p[ci, kd:kd + D, kh:kh + H, kw:kw + W] * w2_ref[idx]
                        acc = t if acc is None else acc + t
        o_ref[...] = acc + b2_ref[0]

    glyph_spec = [
        pl.BlockSpec((1, 50, 50),
                     (lambda n: (lambda b, ids, val: (ids[b, n], 0, 0)))(n))
        for n in range(N)
    ]
    smem = pl.BlockSpec(memory_space=pltpu.MemorySpace.SMEM)

    return pl.pallas_call(
        body,
        out_shape=jax.ShapeDtypeStruct((B, D, H, W), _F32),
        grid_spec=pltpu.PrefetchScalarGridSpec(
            num_scalar_prefetch=2,
            grid=(B,),
            in_specs=glyph_spec + [smem, smem, smem, smem],
            out_specs=pl.BlockSpec((None, D, H, W), lambda b, ids, val: (b, 0, 0, 0)),
            scratch_shapes=[
                pltpu.VMEM((Dp, Hp, Wp), _F32),
                pltpu.VMEM((4, Dp, Hp, Wp), _F32),
            ],
        ),
        compiler_params=_params(),
    )(safe, valid, *([table] * N), w1, b1, w2, b2)


def _pool_matrix(src_w, n_out, dst_w, dst_off, n_in, rows):
    S = np.zeros((rows, n_in), np.float32)
    for i in range(n_out):
        for j in range(n_out):
            S[(i + dst_off) * dst_w + (j + dst_off), 2 * i * src_w + 2 * j] = 1.0
    return S


def _stage_b(x, w2d, b2d, wfc, bfc):
    B = x.shape[0]
    C = 16
    wt = jnp.transpose(w2d, (2, 3, 1, 0)).reshape(9 * C, C).astype(_F32)
    bias = b2d.reshape(1, C).astype(_F32)
    wf = wfc.astype(_F32)
    bf = bfc.reshape(1, -1).astype(_F32)

    L1 = 2598
    M1 = L1 - 53
    L2 = 673
    M2 = L2 - 28
    S1 = jnp.asarray(_pool_matrix(52, 25, 27, 1, M1, 729))
    S2 = jnp.asarray(_pool_matrix(27, 12, 12, 0, M2, 144))

    def body(x_ref, wt_ref, b_ref, s1_ref, s2_ref, wf_ref, bf_ref, o_ref):
        xin = x_ref[0]
        acc = None
        for kh in range(3):
            for kw in range(3):
                t = kh * 3 + kw
                p = jnp.dot(xin[kh * 52 + kw: kh * 52 + kw + L1, :],
                            wt_ref[t * C:(t + 1) * C, :],
                            preferred_element_type=_F32)
                acc = p if acc is None else acc + p
        c1 = acc + b_ref[...]
        m = jnp.maximum(jnp.maximum(c1[0:M1], c1[1:M1 + 1]),
                        jnp.maximum(c1[52:M1 + 52], c1[53:M1 + 53]))
        p1 = jnp.dot(s1_ref[...], m, preferred_element_type=_F32)

        acc = None
        for kh in range(3):
            for kw in range(3):
                t = kh * 3 + kw
                p = jnp.dot(p1[kh * 27 + kw: kh * 27 + kw + L2, :],
                            wt_ref[t * C:(t + 1) * C, :],
                            preferred_element_type=_F32)
                acc = p if acc is None else acc + p
        c2 = acc + b_ref[...]
        m2 = jnp.maximum(jnp.maximum(c2[0:M2], c2[1:M2 + 1]),
                         jnp.maximum(c2[27:M2 + 27], c2[28:M2 + 28]))
        p2 = jnp.dot(s2_ref[...], m2, preferred_element_type=_F32)

        r = jnp.maximum(p2, 0.0)
        out = jax.lax.dot_general(r, wf_ref[...], (((0,), (0,)), ((), ())),
                                  preferred_element_type=_F32)
        o_ref[0] = out + bf_ref[...]

    const = lambda shape: pl.BlockSpec(shape, lambda b: (0,) * len(shape))
    return pl.pallas_call(
        body,
        out_shape=jax.ShapeDtypeStruct((B, C, wf.shape[1]), _F32),
        grid=(B,),
        in_specs=[
            pl.BlockSpec((1, 52 * 52, C), lambda b: (b, 0, 0)),
            const(wt.shape), const(bias.shape), const(S1.shape),
            const(S2.shape), const(wf.shape), const(bf.shape),
        ],
        out_specs=pl.BlockSpec((1, C, wf.shape[1]), lambda b: (b, 0, 0)),
        compiler_params=_params(),
    )(x, wt, bias, S1, S2, wf, bf)


def kernel(char_ids, img_table, w3d1, b3d1, w3d2, b3d2, w2d, b2d, wfc, bfc):
    h2 = _stage_a(char_ids, img_table, w3d1, b3d1, w3d2, b3d2)
    B, N = h2.shape[:2]
    hp = jnp.pad(h2, ((0, 0), (0, 0), (1, 1), (1, 1)))
    hf = jnp.transpose(hp, (0, 2, 3, 1)).reshape(B, 52 * 52, N)
    return _stage_b(hf, w2d, b2d, wfc, bfc)


# speedup vs baseline: 1.4488x; 1.0494x over previous
import numpy as np
import jax
import jax.numpy as jnp
from jax.experimental import pallas as pl
from jax.experimental.pallas import tpu as pltpu

_F32 = jnp.float32


def _params():
    return pltpu.CompilerParams(
        dimension_semantics=("parallel",),
        vmem_limit_bytes=48 * 1024 * 1024,
    )


def _stage_a(char_ids, img_table, w3d1, b3d1, w3d2, b3d2):
    B, N = char_ids.shape
    V = img_table.shape[0]
    table = img_table.reshape(V, 50, 50)
    D, H, W = N, 50, 50
    Dp, Hp, Wp = D + 2, H + 2, W + 2

    ids = char_ids.astype(jnp.int32)
    valid = ((ids >= 0) & (ids < V)).astype(jnp.int32)
    safe = jnp.clip(ids, 0, V - 1)

    w1 = w3d1.reshape(-1).astype(_F32)
    b1 = b3d1.reshape(-1).astype(_F32)
    w2 = w3d2.reshape(-1).astype(_F32)
    b2 = b3d2.reshape(-1).astype(_F32)

    def body(ids_ref, valid_ref, *refs):
        glyph_refs = refs[:N]
        w1_ref, b1_ref, w2_ref, b2_ref, o_ref, xp, h1p = refs[N:]
        b = pl.program_id(0)

        xp[...] = jnp.zeros((Dp, Hp, Wp), _F32)
        for n in range(N):
            v = valid_ref[b, n].astype(_F32)
            xp[n + 1, 1:1 + H, 1:1 + W] = glyph_refs[n][0] * v

        xs = [xp[:, kh:kh + H, kw:kw + W] for kh in range(3) for kw in range(3)]
        h1p[...] = jnp.zeros((4, Dp, Hp, Wp), _F32)
        for co in range(4):
            acc = None
            for kd in range(3):
                for t in range(9):
                    idx = (co * 3 + kd) * 9 + t
                    term = xs[t][kd:kd + D] * w1_ref[idx]
                    acc = term if acc is None else acc + term
            h1p[co, 1:1 + D, 1:1 + H, 1:1 + W] = acc + b1_ref[co]

        acc = None
        for ci in range(4):
            hs = [h1p[ci, :, kh:kh + H, kw:kw + W]
                  for kh in range(3) for kw in range(3)]
            for kd in range(3):
                for t in range(9):
                    idx = (ci * 3 + kd) * 9 + t
                    term = hs[t][kd:kd + D] * w2_ref[idx]
                    acc = term if acc is None else acc + term
        o_ref[...] = jnp.zeros((D, Hp, Wp), _F32)
        o_ref[:, 1:1 + H, 1:1 + W] = acc + b2_ref[0]

    glyph_spec = [
        pl.BlockSpec((1, 50, 50),
                     (lambda n: (lambda b, ids, val: (ids[b, n], 0, 0)))(n))
        for n in range(N)
    ]
    smem = pl.BlockSpec(memory_space=pltpu.MemorySpace.SMEM)

    return pl.pallas_call(
        body,
        out_shape=jax.ShapeDtypeStruct((B, D, Hp, Wp), _F32),
        grid_spec=pltpu.PrefetchScalarGridSpec(
            num_scalar_prefetch=2,
            grid=(B,),
            in_specs=glyph_spec + [smem, smem, smem, smem],
            out_specs=pl.BlockSpec((None, D, Hp, Wp),
                                   lambda b, ids, val: (b, 0, 0, 0)),
            scratch_shapes=[
                pltpu.VMEM((Dp, Hp, Wp), _F32),
                pltpu.VMEM((4, Dp, Hp, Wp), _F32),
            ],
        ),
        compiler_params=_params(),
    )(safe, valid, *([table] * N), w1, b1, w2, b2)


def _pool_matrix_t(src_w, n_out, dst_w, dst_off, n_in, rows):
    S = np.zeros((n_in, rows), np.float32)
    for i in range(n_out):
        for j in range(n_out):
            S[2 * i * src_w + 2 * j, (i + dst_off) * dst_w + (j + dst_off)] = 1.0
    return S


def _stage_b(x, w2d, b2d, wfc, bfc):
    B = x.shape[0]
    C = 16
    wt = jnp.transpose(w2d, (2, 3, 0, 1)).reshape(9 * C, C).astype(_F32)
    bias = b2d.reshape(C, 1).astype(_F32)
    wf = wfc.astype(_F32)
    bf = bfc.reshape(1, -1).astype(_F32)

    L1 = 2598
    M1 = L1 - 53
    L2 = 673
    M2 = L2 - 28
    S1 = jnp.asarray(_pool_matrix_t(52, 25, 27, 1, M1, 729))
    S2 = jnp.asarray(_pool_matrix_t(27, 12, 12, 0, M2, 144))

    def body(x_ref, wt_ref, b_ref, s1_ref, s2_ref, wf_ref, bf_ref, o_ref):
        xin = x_ref[0]
        acc = None
        for kh in range(3):
            for kw in range(3):
                t = kh * 3 + kw
                p = jnp.dot(wt_ref[t * C:(t + 1) * C, :],
                            xin[:, kh * 52 + kw: kh * 52 + kw + L1],
                            preferred_element_type=_F32)
                acc = p if acc is None else acc + p
        c1 = acc + b_ref[...]
        m = jnp.maximum(jnp.maximum(c1[:, 0:M1], c1[:, 1:M1 + 1]),
                        jnp.maximum(c1[:, 52:M1 + 52], c1[:, 53:M1 + 53]))
        p1 = jnp.dot(m, s1_ref[...], preferred_element_type=_F32)

        acc = None
        for kh in range(3):
            for kw in range(3):
                t = kh * 3 + kw
                p = jnp.dot(wt_ref[t * C:(t + 1) * C, :],
                            p1[:, kh * 27 + kw: kh * 27 + kw + L2],
                            preferred_element_type=_F32)
                acc = p if acc is None else acc + p
        c2 = acc + b_ref[...]
        m2 = jnp.maximum(jnp.maximum(c2[:, 0:M2], c2[:, 1:M2 + 1]),
                         jnp.maximum(c2[:, 27:M2 + 27], c2[:, 28:M2 + 28]))
        p2 = jnp.dot(m2, s2_ref[...], preferred_element_type=_F32)

        out = jnp.dot(jnp.maximum(p2, 0.0), wf_ref[...],
                      preferred_element_type=_F32)
        o_ref[0] = out + bf_ref[...]

    const = lambda shape: pl.BlockSpec(shape, lambda b: (0,) * len(shape))
    return pl.pallas_call(
        body,
        out_shape=jax.ShapeDtypeStruct((B, C, wf.shape[1]), _F32),
        grid=(B,),
        in_specs=[
            pl.BlockSpec((1, C, 52 * 52), lambda b: (b, 0, 0)),
            const(wt.shape), const(bias.shape), const(S1.shape),
            const(S2.shape), const(wf.shape), const(bf.shape),
        ],
        out_specs=pl.BlockSpec((1, C, wf.shape[1]), lambda b: (b, 0, 0)),
        compiler_params=_params(),
    )(x, wt, bias, S1, S2, wf, bf)


def kernel(char_ids, img_table, w3d1, b3d1, w3d2, b3d2, w2d, b2d, wfc, bfc):
    hp = _stage_a(char_ids, img_table, w3d1, b3d1, w3d2, b3d2)
    B, N = hp.shape[:2]
    hf = hp.reshape(B, N, 52 * 52)
    return _stage_b(hf, w2d, b2d, wfc, bfc)


# speedup vs baseline: 2.7563x; 1.9024x over previous
import numpy as np
import jax
import jax.numpy as jnp
from jax.experimental import pallas as pl
from jax.experimental.pallas import tpu as pltpu

_F32 = jnp.float32

_PW = 52
_FL = _PW * _PW
_L1 = 2598
_M1 = _L1 - 53
_L2 = 673
_M2 = _L2 - 28


def _pool_matrix_t(src_w, n_out, dst_w, dst_off, n_in, rows):
    S = np.zeros((n_in, rows), np.float32)
    for i in range(n_out):
        for j in range(n_out):
            S[2 * i * src_w + 2 * j, (i + dst_off) * dst_w + (j + dst_off)] = 1.0
    return S


def kernel(char_ids, img_table, w3d1, b3d1, w3d2, b3d2, w2d, b2d, wfc, bfc):
    B, N = char_ids.shape
    V = img_table.shape[0]
    D = N
    hidden = wfc.shape[1]

    tpad = jnp.pad(img_table.reshape(V, 50, 50).astype(_F32),
                   ((0, 0), (1, 1), (1, 1))).reshape(V, 1, _FL)
    ids = char_ids.astype(jnp.int32)
    valid = ((ids >= 0) & (ids < V)).astype(jnp.int32)
    safe = jnp.clip(ids, 0, V - 1)

    w1 = w3d1.reshape(-1).astype(_F32)
    b1 = b3d1.reshape(-1).astype(_F32)
    w2 = w3d2.reshape(-1).astype(_F32)
    b2 = b3d2.reshape(-1).astype(_F32)
    wt = jnp.transpose(w2d, (2, 3, 0, 1)).reshape(9 * D, D).astype(_F32)
    b2d_c = b2d.reshape(D, 1).astype(_F32)
    wf = wfc.astype(_F32)
    bf = bfc.reshape(1, hidden).astype(_F32)
    S1 = jnp.asarray(_pool_matrix_t(52, 25, 27, 1, _M1, 729))
    S2 = jnp.asarray(_pool_matrix_t(27, 12, 12, 0, _M2, 144))

    colmask = jnp.asarray((np.arange(_L1) % _PW < 50).astype(np.float32))

    def body(ids_ref, valid_ref, *refs):
        glyph_refs = refs[:N]
        (w1_ref, b1_ref, w2_ref, b2_ref, wt_ref, bc_ref, s1_ref, s2_ref,
         wf_ref, bf_ref, cm_ref, o_ref, xp, h1p, x2) = refs[N:]
        b = pl.program_id(0)

        xp[...] = jnp.zeros((D + 2, _FL), _F32)
        for n in range(N):
            v = valid_ref[b, n].astype(_F32)
            xp[n + 1:n + 2, :] = glyph_refs[n][0] * v

        cmask = cm_ref[...]

        xs = [xp[kd:kd + D, kh * _PW + kw: kh * _PW + kw + _L1]
              for kd in range(3) for kh in range(3) for kw in range(3)]
        h1p[...] = jnp.zeros((4, D + 2, _FL), _F32)
        for co in range(4):
            acc = None
            for t in range(27):
                term = xs[t] * w1_ref[co * 27 + t]
                acc = term if acc is None else acc + term
            h1p[co, 1:1 + D, 53:53 + _L1] = (acc + b1_ref[co]) * cmask

        acc = None
        for ci in range(4):
            for t in range(27):
                kd, r = divmod(t, 9)
                kh, kw = divmod(r, 3)
                term = (h1p[ci, kd:kd + D, kh * _PW + kw: kh * _PW + kw + _L1]
                        * w2_ref[ci * 27 + t])
                acc = term if acc is None else acc + term
        x2[...] = jnp.zeros((D, _FL), _F32)
        x2[:, 53:53 + _L1] = (acc + b2_ref[0]) * cmask

        xin = x2[...]
        acc = None
        for t in range(9):
            kh, kw = divmod(t, 3)
            p = jnp.dot(wt_ref[t * D:(t + 1) * D, :],
                        xin[:, kh * _PW + kw: kh * _PW + kw + _L1],
                        preferred_element_type=_F32)
            acc = p if acc is None else acc + p
        c1 = acc + bc_ref[...]
        m = jnp.maximum(jnp.maximum(c1[:, 0:_M1], c1[:, 1:_M1 + 1]),
                        jnp.maximum(c1[:, 52:_M1 + 52], c1[:, 53:_M1 + 53]))
        p1 = jnp.dot(m, s1_ref[...], preferred_element_type=_F32)

        acc = None
        for t in range(9):
            kh, kw = divmod(t, 3)
            p = jnp.dot(wt_ref[t * D:(t + 1) * D, :],
                        p1[:, kh * 27 + kw: kh * 27 + kw + _L2],
                        preferred_element_type=_F32)
            acc = p if acc is None else acc + p
        c2 = acc + bc_ref[...]
        m2 = jnp.maximum(jnp.maximum(c2[:, 0:_M2], c2[:, 1:_M2 + 1]),
                         jnp.maximum(c2[:, 27:_M2 + 27], c2[:, 28:_M2 + 28]))
        p2 = jnp.dot(m2, s2_ref[...], preferred_element_type=_F32)

        out = jnp.dot(jnp.maximum(p2, 0.0), wf_ref[...],
                      preferred_element_type=_F32)
        o_ref[0] = out + bf_ref[...]

    glyph_spec = [
        pl.BlockSpec((1, 1, _FL),
                     (lambda n: (lambda b, ids, val: (ids[b, n], 0, 0)))(n))
        for n in range(N)
    ]
    smem = pl.BlockSpec(memory_space=pltpu.MemorySpace.SMEM)
    const = lambda shape: pl.BlockSpec(shape, lambda b, ids, val: (0,) * len(shape))

    return pl.pallas_call(
        body,
        out_shape=jax.ShapeDtypeStruct((B, D, hidden), _F32),
        grid_spec=pltpu.PrefetchScalarGridSpec(
            num_scalar_prefetch=2,
            grid=(B,),
            in_specs=glyph_spec + [smem, smem, smem, smem,
                                   const(wt.shape), const(b2d_c.shape),
                                   const(S1.shape), const(S2.shape),
                                   const(wf.shape), const(bf.shape),
                                   const((1, _L1))],
            out_specs=pl.BlockSpec((1, D, hidden), lambda b, ids, val: (b, 0, 0)),
            scratch_shapes=[
                pltpu.VMEM((D + 2, _FL), _F32),
                pltpu.VMEM((4, D + 2, _FL), _F32),
                pltpu.VMEM((D, _FL), _F32),
            ],
        ),
        compiler_params=pltpu.CompilerParams(
            dimension_semantics=("parallel",),
            vmem_limit_bytes=48 * 1024 * 1024,
        ),
    )(safe, valid, *([tpad] * N), w1, b1, w2, b2, wt, b2d_c, S1, S2,
      wf, bf, colmask.reshape(1, _L1))


# speedup vs baseline: 4.8755x; 1.7689x over previous
import numpy as np
import jax
import jax.numpy as jnp
from jax.experimental import pallas as pl
from jax.experimental.pallas import tpu as pltpu

_F32 = jnp.float32

_PW = 52
_FL = _PW * _PW
_L1 = 2598
_M1 = _L1 - 53
_L2 = 673
_M2 = _L2 - 28


def _pool_matrix_t(src_w, n_out, dst_w, dst_off, n_in, rows):
    S = np.zeros((n_in, rows), np.float32)
    for i in range(n_out):
        for j in range(n_out):
            S[2 * i * src_w + 2 * j, (i + dst_off) * dst_w + (j + dst_off)] = 1.0
    return S


def kernel(char_ids, img_table, w3d1, b3d1, w3d2, b3d2, w2d, b2d, wfc, bfc):
    B, N = char_ids.shape
    V = img_table.shape[0]
    D = N
    hidden = wfc.shape[1]

    tpad = jnp.pad(img_table.reshape(V, 50, 50).astype(_F32),
                   ((0, 0), (1, 1), (1, 1))).reshape(V, 1, _FL)
    ids = char_ids.astype(jnp.int32)
    valid = ((ids >= 0) & (ids < V)).astype(jnp.int32)
    safe = jnp.clip(ids, 0, V - 1)

    w1 = w3d1.reshape(-1).astype(_F32)
    b1 = b3d1.reshape(-1).astype(_F32)
    w2 = w3d2.reshape(-1).astype(_F32)
    b2 = b3d2.reshape(-1).astype(_F32)
    wt = jnp.transpose(w2d, (2, 3, 0, 1)).reshape(9 * D, D).astype(_F32)
    b2d_c = b2d.reshape(D, 1).astype(_F32)
    wf = wfc.astype(_F32)
    bf = bfc.reshape(1, hidden).astype(_F32)
    S1 = jnp.asarray(_pool_matrix_t(52, 25, 27, 1, _M1, 729))
    S2 = jnp.asarray(_pool_matrix_t(27, 12, 12, 0, _M2, 144))

    colmask = jnp.asarray((np.arange(_L1) % _PW < 50).astype(np.float32))

    def body(ids_ref, valid_ref, *refs):
        glyph_refs = refs[:N]
        (w1_ref, b1_ref, w2_ref, b2_ref, wt_ref, bc_ref, s1_ref, s2_ref,
         wf_ref, bf_ref, cm_ref, o_ref, xps, h1s, x2) = refs[N:]
        b = pl.program_id(0)

        xps[...] = jnp.zeros((3, D, _FL), _F32)
        for n in range(N):
            v = valid_ref[b, n].astype(_F32)
            row = glyph_refs[n][0] * v
            for k in range(3):
                j = n + 1 - k
                if 0 <= j < D:
                    xps[k, j:j + 1, :] = row

        cmask = cm_ref[...]

        h1s[...] = jnp.zeros((3, 4, D, _FL), _F32)
        for co in range(4):
            acc = None
            for t in range(9):
                kh, kw = divmod(t, 3)
                g = None
                for kd in range(3):
                    term = xps[kd] * w1_ref[(co * 3 + kd) * 9 + t]
                    g = term if g is None else g + term
                sl = g[:, kh * _PW + kw: kh * _PW + kw + _L1]
                acc = sl if acc is None else acc + sl
            am = (acc + b1_ref[co]) * cmask
            h1s[0, co, 1:D, 53:53 + _L1] = am[0:D - 1]
            h1s[1, co, :, 53:53 + _L1] = am
            h1s[2, co, 0:D - 1, 53:53 + _L1] = am[1:D]

        acc = None
        for t in range(9):
            kh, kw = divmod(t, 3)
            g = None
            for ci in range(4):
                for kd in range(3):
                    term = h1s[kd, ci] * w2_ref[(ci * 3 + kd) * 9 + t]
                    g = term if g is None else g + term
            sl = g[:, kh * _PW + kw: kh * _PW + kw + _L1]
            acc = sl if acc is None else acc + sl
        x2[...] = jnp.zeros((D, _FL), _F32)
        x2[:, 53:53 + _L1] = (acc + b2_ref[0]) * cmask

        xin = x2[...]
        acc = None
        for t in range(9):
            kh, kw = divmod(t, 3)
            p = jnp.dot(wt_ref[t * D:(t + 1) * D, :],
                        xin[:, kh * _PW + kw: kh * _PW + kw + _L1],
                        preferred_element_type=_F32)
            acc = p if acc is None else acc + p
        c1 = acc + bc_ref[...]
        m = jnp.maximum(jnp.maximum(c1[:, 0:_M1], c1[:, 1:_M1 + 1]),
                        jnp.maximum(c1[:, 52:_M1 + 52], c1[:, 53:_M1 + 53]))
        p1 = jnp.dot(m, s1_ref[...], preferred_element_type=_F32)

        acc = None
        for t in range(9):
            kh, kw = divmod(t, 3)
            p = jnp.dot(wt_ref[t * D:(t + 1) * D, :],
                        p1[:, kh * 27 + kw: kh * 27 + kw + _L2],
                        preferred_element_type=_F32)
            acc = p if acc is None else acc + p
        c2 = acc + bc_ref[...]
        m2 = jnp.maximum(jnp.maximum(c2[:, 0:_M2], c2[:, 1:_M2 + 1]),
                         jnp.maximum(c2[:, 27:_M2 + 27], c2[:, 28:_M2 + 28]))
        p2 = jnp.dot(m2, s2_ref[...], preferred_element_type=_F32)

        out = jnp.dot(jnp.maximum(p2, 0.0), wf_ref[...],
                      preferred_element_type=_F32)
        o_ref[0] = out + bf_ref[...]

    glyph_spec = [
        pl.BlockSpec((1, 1, _FL),
                     (lambda n: (lambda b, ids, val: (ids[b, n], 0, 0)))(n))
        for n in range(N)
    ]
    smem = pl.BlockSpec(memory_space=pltpu.MemorySpace.SMEM)
    const = lambda shape: pl.BlockSpec(shape, lambda b, ids, val: (0,) * len(shape))

    return pl.pallas_call(
        body,
        out_shape=jax.ShapeDtypeStruct((B, D, hidden), _F32),
        grid_spec=pltpu.PrefetchScalarGridSpec(
            num_scalar_prefetch=2,
            grid=(B,),
            in_specs=glyph_spec + [smem, smem, smem, smem,
                                   const(wt.shape), const(b2d_c.shape),
                                   const(S1.shape), const(S2.shape),
                                   const(wf.shape), const(bf.shape),
                                   const((1, _L1))],
            out_specs=pl.BlockSpec((1, D, hidden), lambda b, ids, val: (b, 0, 0)),
            scratch_shapes=[
                pltpu.VMEM((3, D, _FL), _F32),
                pltpu.VMEM((3, 4, D, _FL), _F32),
                pltpu.VMEM((D, _FL), _F32),
            ],
        ),
        compiler_params=pltpu.CompilerParams(
            dimension_semantics=("parallel",),
            vmem_limit_bytes=48 * 1024 * 1024,
        ),
    )(safe, valid, *([tpad] * N), w1, b1, w2, b2, wt, b2d_c, S1, S2,
      wf, bf, colmask.reshape(1, _L1))
